# Initial kernel scaffold; baseline (speedup 1.0000x reference)
import jax
import jax.numpy as jnp
from jax.experimental import pallas as pl


def kernel(x, edges, kernel, bias):
    raise NotImplementedError("write your pallas kernel here")


# speedup vs baseline: 1.5281x; 1.5281x over previous
import jax
import jax.numpy as jnp
from jax import lax
from jax.experimental import pallas as pl
from jax.experimental.pallas import tpu as pltpu
from jax.experimental.pallas import tpu_sc as plsc

B = 256
N_IN = 10000
N_OUT = 4096
N_CONN = 32
LANES = 16


def _make_sc_kernel(n_in, n_out, b, n_conn, num_cores, num_subcores,
                    interpret=False):
    nw = num_cores * num_subcores
    out_per_w = n_out // nw
    nchunk = b // LANES
    wrow = n_conn * LANES
    mesh = plsc.VectorSubcoreMesh(core_axis_name="c", subcore_axis_name="s",
                                  num_cores=num_cores,
                                  num_subcores=num_subcores)

    def body(xT, edges, wbc, bbc, out, edges_v, wbc_v, bbc_v, rows_v, out_v):
        wid = lax.axis_index("s") * num_cores + lax.axis_index("c")
        base = wid * out_per_w
        pltpu.sync_copy(edges.at[pl.ds(base * n_conn, out_per_w * n_conn)],
                        edges_v)
        pltpu.sync_copy(wbc.at[pl.ds(base * wrow, out_per_w * wrow)], wbc_v)
        pltpu.sync_copy(bbc.at[pl.ds(base * LANES, out_per_w * LANES)], bbc_v)

        def outer(o, carry):
            pltpu.sync_copy(xT.at[edges_v.at[pl.ds(o * n_conn, n_conn)]],
                            rows_v)
            bias_vec = bbc_v[pl.ds(o * LANES, LANES)]
            accs = [bias_vec] * nchunk
            for c in range(n_conn):
                w = wbc_v[pl.ds(o * wrow + c * LANES, LANES)]
                for k in range(nchunk):
                    accs[k] = accs[k] + rows_v[c, pl.ds(k * LANES, LANES)] * w
            for k in range(nchunk):
                out_v[o, pl.ds(k * LANES, LANES)] = accs[k]
            return carry

        lax.fori_loop(0, out_per_w, outer, 0)
        pltpu.sync_copy(out_v, out.at[pl.ds(base, out_per_w)])

    return pl.kernel(
        body,
        out_type=jax.ShapeDtypeStruct((n_out, b), jnp.float32),
        mesh=mesh,
        scratch_types=[
            pltpu.VMEM((out_per_w * n_conn,), jnp.int32),
            pltpu.VMEM((out_per_w * wrow,), jnp.float32),
            pltpu.VMEM((out_per_w * LANES,), jnp.float32),
            pltpu.VMEM((n_conn, b), jnp.float32),
            pltpu.VMEM((out_per_w, b), jnp.float32),
        ],
        interpret=interpret,
    )


@jax.jit
def kernel(x, edges, kernel, bias):
    xT = x.T
    wbc = jnp.broadcast_to(kernel[:, :, None],
                           (N_OUT, N_CONN, LANES)).reshape(-1)
    bbc = jnp.broadcast_to(bias[:, None], (N_OUT, LANES)).reshape(-1)
    sc = _make_sc_kernel(N_IN, N_OUT, B, N_CONN, 2, 16)
    yT = sc(xT, edges.reshape(-1), wbc, bbc)
    return yT.T


# speedup vs baseline: 1.9029x; 1.2452x over previous
import jax
import jax.numpy as jnp
from jax import lax
from jax.experimental import pallas as pl
from jax.experimental.pallas import tpu as pltpu
from jax.experimental.pallas import tpu_sc as plsc

B = 256
N_IN = 10000
N_OUT = 4096
N_CONN = 32
LANES = 16


def _make_sc_kernel(n_in, n_out, b, n_conn, num_cores, num_subcores,
                    interpret=False):
    nw = num_cores * num_subcores
    out_per_w = n_out // nw
    nchunk = b // LANES
    wrow = n_conn * LANES
    mesh = plsc.VectorSubcoreMesh(core_axis_name="c", subcore_axis_name="s",
                                  num_cores=num_cores,
                                  num_subcores=num_subcores)

    def body(xT, edges, wbc, bbc, out, edges_v, wbc_v, bbc_v, rows_v, out_v,
             sem0, sem1):
        wid = lax.axis_index("s") * num_cores + lax.axis_index("c")
        base = wid * out_per_w
        pltpu.sync_copy(edges.at[pl.ds(base * n_conn, out_per_w * n_conn)],
                        edges_v)
        pltpu.sync_copy(wbc.at[pl.ds(base * wrow, out_per_w * wrow)], wbc_v)
        pltpu.sync_copy(bbc.at[pl.ds(base * LANES, out_per_w * LANES)], bbc_v)

        def gather(o, buf, sem):
            idx = edges_v.at[pl.ds(o * n_conn, n_conn)]
            return pltpu.make_async_copy(xT.at[idx], rows_v.at[buf], sem)

        def compute(o, buf):
            bias_vec = bbc_v[pl.ds(o * LANES, LANES)]
            accs = [bias_vec] * nchunk
            for c in range(n_conn):
                w = wbc_v[pl.ds(o * wrow + c * LANES, LANES)]
                for k in range(nchunk):
                    accs[k] = (accs[k]
                               + rows_v[buf, c, pl.ds(k * LANES, LANES)] * w)
            for k in range(nchunk):
                out_v[o, pl.ds(k * LANES, LANES)] = accs[k]

        gather(0, 0, sem0).start()
        gather(1, 1, sem1).start()

        def outer(o2, carry):
            o = o2 * 2
            gather(o, 0, sem0).wait()
            compute(o, 0)

            @pl.when(o + 2 < out_per_w)
            def _():
                gather(o + 2, 0, sem0).start()

            gather(o + 1, 1, sem1).wait()
            compute(o + 1, 1)

            @pl.when(o + 3 < out_per_w)
            def _():
                gather(o + 3, 1, sem1).start()

            return carry

        lax.fori_loop(0, out_per_w // 2, outer, 0)
        pltpu.sync_copy(out_v, out.at[pl.ds(base, out_per_w)])

    return pl.kernel(
        body,
        out_type=jax.ShapeDtypeStruct((n_out, b), jnp.float32),
        mesh=mesh,
        scratch_types=[
            pltpu.VMEM((out_per_w * n_conn,), jnp.int32),
            pltpu.VMEM((out_per_w * wrow,), jnp.float32),
            pltpu.VMEM((out_per_w * LANES,), jnp.float32),
            pltpu.VMEM((2, n_conn, b), jnp.float32),
            pltpu.VMEM((out_per_w, b), jnp.float32),
            pltpu.SemaphoreType.DMA,
            pltpu.SemaphoreType.DMA,
        ],
        interpret=interpret,
    )


@jax.jit
def kernel(x, edges, kernel, bias):
    xT = x.T
    wbc = jnp.broadcast_to(kernel[:, :, None],
                           (N_OUT, N_CONN, LANES)).reshape(-1)
    bbc = jnp.broadcast_to(bias[:, None], (N_OUT, LANES)).reshape(-1)
    sc = _make_sc_kernel(N_IN, N_OUT, B, N_CONN, 2, 16)
    yT = sc(xT, edges.reshape(-1), wbc, bbc)
    return yT.T
